# baseline (speedup 1.0000x reference)
import functools

import jax
import jax.numpy as jnp
from jax import lax
from jax.experimental import pallas as pl
from jax.experimental.pallas import tpu as pltpu
from jax.experimental.pallas import tpu_sc as plsc

B = 16384
NROWS = 1_000_000
D = 64
NC = 2
NS = 16
L = 16
N_TILE = B // NS
N_FIN = B // (NC * NS)
CHUNK = 128
NCHUNK = N_TILE // CHUNK
NFCHUNK = N_FIN // CHUNK
NBATCH = 2
BCH = NFCHUNK // NBATCH
BPOS = BCH * CHUNK
DUMMY = NROWS
TBL = NROWS + 8
ROUNDS = 4

_mesh = plsc.VectorSubcoreMesh(
    core_axis_name="c", subcore_axis_name="s", num_cores=NC
)


@functools.partial(
    pl.kernel,
    out_type=(
        jax.ShapeDtypeStruct((B // CHUNK, CHUNK), jnp.int32),
        jax.ShapeDtypeStruct((B // CHUNK, CHUNK), jnp.int32),
        jax.ShapeDtypeStruct((B // CHUNK, CHUNK), jnp.int32),
        jax.ShapeDtypeStruct((B // CHUNK, CHUNK), jnp.int32),
    ),
    mesh=_mesh,
    compiler_params=pltpu.CompilerParams(use_tc_tiling_on_sc=False),
    scratch_types=[
        pltpu.VMEM_SHARED((TBL,), jnp.int32),
        pltpu.VMEM_SHARED((B,), jnp.int32),
        pltpu.VMEM((NCHUNK, CHUNK), jnp.int32),
        pltpu.VMEM((NCHUNK, CHUNK), jnp.int32),
        pltpu.VMEM((NCHUNK, CHUNK), jnp.int32),
        pltpu.VMEM((NCHUNK, CHUNK), jnp.int32),
        pltpu.VMEM((NCHUNK, CHUNK), jnp.int32),
        pltpu.VMEM((NCHUNK, CHUNK), jnp.int32),
        pltpu.VMEM((NCHUNK, CHUNK), jnp.int32),
        pltpu.SemaphoreType.DMA,
    ],
)
def _sc_dedup(idx_hbm, code_hbm, pidx_hbm, wrow_hbm, prow_hbm, tbl, idx_sh,
              idx_v, pos_v, sidx_v, w_v, pwin_v, prow_v, code_v, sem):
    c = lax.axis_index("c")
    s = lax.axis_index("s")
    base = s * N_TILE
    lane = lax.iota(jnp.int32, L)

    pltpu.sync_copy(idx_hbm.at[pl.ds(s * NCHUNK, NCHUNK)], idx_v)
    for j in range(NCHUNK):
        pltpu.sync_copy(idx_v.at[j], idx_sh.at[pl.ds(base + j * CHUNK, CHUNK)])
    for j in range(NCHUNK):
        for k in range(CHUNK // L):
            pos_v[j, pl.ds(k * L, L)] = base + (j * CHUNK + k * L) + lane

    def _scatter_pos(index_ref):
        cps = [pltpu.async_copy(pos_v.at[j], tbl.at[index_ref.at[j]], sem)
               for j in range(NCHUNK)]
        for c_ in cps:
            c_.wait()

    def _gather_w():
        cps = [pltpu.async_copy(tbl.at[idx_v.at[j]], w_v.at[j], sem)
               for j in range(NCHUNK)]
        for c_ in cps:
            c_.wait()

    _scatter_pos(idx_v)
    plsc.subcore_barrier()
    _gather_w()

    def _round(_, carry):
        for j in range(NCHUNK):
            for k in range(CHUNK // L):
                sl = pl.ds(k * L, L)
                p = pos_v[j, sl]
                w = w_v[j, sl]
                sidx_v[j, sl] = jnp.where(p > w, idx_v[j, sl], DUMMY)
        plsc.subcore_barrier()
        _scatter_pos(sidx_v)
        plsc.subcore_barrier()
        _gather_w()
        return carry

    lax.fori_loop(0, ROUNDS, _round, 0)
    plsc.subcore_barrier()

    for j in range(NCHUNK):
        for k in range(CHUNK // L):
            sl = pl.ds(k * L, L)
            sidx_v[j, sl] = idx_v[j, sl] ^ 1
    cps = [pltpu.async_copy(tbl.at[sidx_v.at[j]], pwin_v.at[j], sem)
           for j in range(NCHUNK)]
    for c_ in cps:
        c_.wait()
    for j in range(NCHUNK):
        for k in range(CHUNK // L):
            sl = pl.ds(k * L, L)
            pw = pwin_v[j, sl]
            pwc = jnp.minimum(jnp.maximum(pw, 0), B - 1)
            pwin_v[j, sl] = pwc
            code_v[j, sl] = jnp.where(pw == pwc, 1, 0)
    cps = [pltpu.async_copy(idx_sh.at[pwin_v.at[j]], prow_v.at[j], sem)
           for j in range(NCHUNK)]
    for c_ in cps:
        c_.wait()
    for j in range(NCHUNK):
        for k in range(CHUNK // L):
            sl = pl.ds(k * L, L)
            r = idx_v[j, sl]
            w = w_v[j, sl]
            pw = pwin_v[j, sl]
            valid = (code_v[j, sl] == 1) & (prow_v[j, sl] == (r ^ 1))
            code_v[j, sl] = ((r & 1) | ((w & 1) << 1) | ((pw & 1) << 2)
                             | (jnp.where(valid, 1, 0) << 3))
            sidx_v[j, sl] = lax.shift_right_logical(r, 1)
            w_v[j, sl] = lax.shift_right_logical(w, 1)
            pwin_v[j, sl] = lax.shift_right_logical(pw, 1)

    @pl.when(c == 0)
    def _():
        pltpu.sync_copy(code_v, code_hbm.at[pl.ds(s * NCHUNK, NCHUNK)])
        pltpu.sync_copy(sidx_v, pidx_hbm.at[pl.ds(s * NCHUNK, NCHUNK)])
        pltpu.sync_copy(w_v, wrow_hbm.at[pl.ds(s * NCHUNK, NCHUNK)])
        pltpu.sync_copy(pwin_v, prow_hbm.at[pl.ds(s * NCHUNK, NCHUNK)])

@functools.partial(
    pl.kernel,
    mesh=_mesh,
    scratch_types=[
        pltpu.VMEM((NCHUNK, CHUNK), jnp.int32),
        pltpu.VMEM((NCHUNK, CHUNK), jnp.int32),
        pltpu.VMEM((NCHUNK, CHUNK), jnp.int32),
        pltpu.VMEM((NCHUNK, CHUNK), jnp.int32),
        pltpu.VMEM((BPOS, 2 * D), jnp.float32),
        pltpu.VMEM((BPOS, 2 * D), jnp.float32),
        pltpu.VMEM((BPOS, 2 * D), jnp.float32),
        pltpu.SemaphoreType.DMA,
    ],
)
def _sc_compose(out2_ref, code_hbm, pidx_hbm, wrow_hbm, prow_hbm, upd2_hbm,
                code_v, pidx_v, wrow_v, prow_v, orig_v, uself_v,
                upart_v, sem):
    c = lax.axis_index("c")
    s = lax.axis_index("s")

    r0 = pl.multiple_of(s * NCHUNK, NCHUNK)
    pltpu.sync_copy(code_hbm.at[pl.ds(r0, NCHUNK)], code_v)
    pltpu.sync_copy(pidx_hbm.at[pl.ds(r0, NCHUNK)], pidx_v)
    pltpu.sync_copy(wrow_hbm.at[pl.ds(r0, NCHUNK)], wrow_v)
    pltpu.sync_copy(prow_hbm.at[pl.ds(r0, NCHUNK)], prow_v)

    fo = c * NFCHUNK

    for b in range(NBATCH):
        j0 = fo + b * BCH
        cps = [pltpu.async_copy(out2_ref.at[pidx_v.at[j0 + j]],
                                orig_v.at[pl.ds(j * CHUNK, CHUNK)], sem)
               for j in range(BCH)]
        cps += [pltpu.async_copy(upd2_hbm.at[wrow_v.at[j0 + j]],
                                 uself_v.at[pl.ds(j * CHUNK, CHUNK)], sem)
                for j in range(BCH)]
        cps += [pltpu.async_copy(upd2_hbm.at[prow_v.at[j0 + j]],
                                 upart_v.at[pl.ds(j * CHUNK, CHUNK)], sem)
                for j in range(BCH)]
        for c_ in cps:
            c_.wait()

        def compose(g, carry):
            gi0 = j0 * CHUNK + g * L
            cv = code_v[gi0 // CHUNK, pl.ds(pl.multiple_of((g * L) % CHUNK, L),
                                            L)]
            for lane in range(L):
                i = g * L + lane
                code = cv[lane]
                po = code & 1
                ps = lax.shift_right_logical(code, 1) & 1
                pp = lax.shift_right_logical(code, 2) & 1
                pv = lax.shift_right_logical(code, 3) & 1
                own_dst = pl.multiple_of(po * D, D)
                own_src = pl.multiple_of(ps * D, D)
                par_dst = pl.multiple_of((1 - po) * D, D)
                par_src = pl.multiple_of(pp * D, D)
                for k in range(D // L):
                    orig_v[i, pl.ds(own_dst + k * L, L)] = (
                        uself_v[i, pl.ds(own_src + k * L, L)])

                @pl.when(pv == 1)
                def _():
                    for k in range(D // L):
                        orig_v[i, pl.ds(par_dst + k * L, L)] = (
                            upart_v[i, pl.ds(par_src + k * L, L)])
            return carry

        lax.fori_loop(0, BPOS // L, compose, 0)

        cps = [pltpu.async_copy(orig_v.at[pl.ds(j * CHUNK, CHUNK)],
                                out2_ref.at[pidx_v.at[j0 + j]], sem)
               for j in range(BCH)]
        for c_ in cps:
            c_.wait()

def kernel(data, indices, updates):
    idx2d = indices.reshape(B).astype(jnp.int32).reshape(B // CHUNK, CHUNK)
    code2d, pidx2d, wrow2d, prow2d = _sc_dedup(idx2d)
    data2 = data.reshape(NROWS // 2, 2 * D)
    upd2 = updates.reshape(B // 2, 2 * D)
    data_ref = jax.new_ref(data2)
    _sc_compose(data_ref, code2d, pidx2d, wrow2d, prow2d, upd2)
    return jax.freeze(data_ref).reshape(NROWS, D)


# speedup vs baseline: 1.2287x; 1.2287x over previous
import functools

import jax
import jax.numpy as jnp
from jax import lax
from jax.experimental import pallas as pl
from jax.experimental.pallas import tpu as pltpu
from jax.experimental.pallas import tpu_sc as plsc

B = 16384
NROWS = 1_000_000
D = 64
NC = 2
NS = 16
L = 16
N_TILE = B // NS
N_FIN = B // (NC * NS)
CHUNK = 128
NCHUNK = N_TILE // CHUNK
NFCHUNK = N_FIN // CHUNK
DUMMY = NROWS
TBL = NROWS + 8
ROUNDS = 4

_mesh = plsc.VectorSubcoreMesh(
    core_axis_name="c", subcore_axis_name="s", num_cores=NC
)


@functools.partial(
    pl.kernel,
    mesh=_mesh,
    compiler_params=pltpu.CompilerParams(use_tc_tiling_on_sc=False),
    scratch_types=[
        pltpu.VMEM_SHARED((TBL,), jnp.int32),
        pltpu.VMEM((NCHUNK, CHUNK), jnp.int32),
        pltpu.VMEM((NCHUNK, CHUNK), jnp.int32),
        pltpu.VMEM((NCHUNK, CHUNK), jnp.int32),
        pltpu.VMEM((NCHUNK, CHUNK), jnp.int32),
        pltpu.VMEM((NFCHUNK, CHUNK), jnp.int32),
        pltpu.VMEM((NFCHUNK, CHUNK), jnp.int32),
        pltpu.VMEM((N_FIN, D), jnp.float32),
        pltpu.SemaphoreType.DMA,
    ],
)
def _sc_scatter(out_ref, idx_hbm, upd_hbm, tbl, idx_v, pos_v, sidx_v,
                w_v, fidx_v, fw_v, frows_v, sem):
    c = lax.axis_index("c")
    s = lax.axis_index("s")
    base = s * N_TILE
    lane = lax.iota(jnp.int32, L)

    pltpu.sync_copy(idx_hbm.at[pl.ds(s * NCHUNK, NCHUNK)], idx_v)
    fin_base = c * (B // NC) + s * N_FIN
    pltpu.sync_copy(idx_hbm.at[pl.ds(fin_base // CHUNK, NFCHUNK)], fidx_v)
    for j in range(NCHUNK):
        for k in range(CHUNK // L):
            pos_v[j, pl.ds(k * L, L)] = base + (j * CHUNK + k * L) + lane

    def _scatter_pos(index_ref):
        cps = [pltpu.async_copy(pos_v.at[j], tbl.at[index_ref.at[j]], sem)
               for j in range(NCHUNK)]
        for c_ in cps:
            c_.wait()

    def _gather_w():
        cps = [pltpu.async_copy(tbl.at[idx_v.at[j]], w_v.at[j], sem)
               for j in range(NCHUNK)]
        for c_ in cps:
            c_.wait()

    _scatter_pos(idx_v)
    plsc.subcore_barrier()
    _gather_w()

    for _ in range(ROUNDS):
        for j in range(NCHUNK):
            for k in range(CHUNK // L):
                sl = pl.ds(k * L, L)
                p = pos_v[j, sl]
                w = w_v[j, sl]
                sidx_v[j, sl] = jnp.where(p > w, idx_v[j, sl], DUMMY)
        plsc.subcore_barrier()
        _scatter_pos(sidx_v)
        plsc.subcore_barrier()
        _gather_w()
    plsc.subcore_barrier()

    cps = [pltpu.async_copy(tbl.at[fidx_v.at[j]], fw_v.at[j], sem)
           for j in range(NFCHUNK)]
    for c_ in cps:
        c_.wait()
    cps = [pltpu.async_copy(upd_hbm.at[fw_v.at[j]],
                            frows_v.at[pl.ds(j * CHUNK, CHUNK)], sem)
           for j in range(NFCHUNK)]
    for c_ in cps:
        c_.wait()
    cps = [pltpu.async_copy(frows_v.at[pl.ds(j * CHUNK, CHUNK)],
                            out_ref.at[fidx_v.at[j]], sem)
           for j in range(NFCHUNK)]
    for c_ in cps:
        c_.wait()


def kernel(data, indices, updates):
    idx = indices.reshape(B).astype(jnp.int32).reshape(B // CHUNK, CHUNK)
    data_ref = jax.new_ref(data)
    _sc_scatter(data_ref, idx, updates)
    return jax.freeze(data_ref)


# speedup vs baseline: 1.6176x; 1.3164x over previous
import functools

import jax
import jax.numpy as jnp
from jax import lax
from jax.experimental import pallas as pl
from jax.experimental.pallas import tpu as pltpu
from jax.experimental.pallas import tpu_sc as plsc

B = 16384
NROWS = 1_000_000
D = 64
NC = 2
NS = 16
L = 16
N_TILE = B // NS
CHUNK = 128
NCHUNK = N_TILE // CHUNK
DUMMY = NROWS
TBL = NROWS + 8
ROUNDS = 4
RING = 128

_mesh = plsc.VectorSubcoreMesh(
    core_axis_name="c", subcore_axis_name="s", num_cores=NC
)


@functools.partial(
    pl.kernel,
    out_type=jax.ShapeDtypeStruct((B // CHUNK, CHUNK), jnp.int32),
    mesh=_mesh,
    compiler_params=pltpu.CompilerParams(use_tc_tiling_on_sc=False),
    scratch_types=[
        pltpu.VMEM_SHARED((TBL,), jnp.int32),
        pltpu.VMEM((NCHUNK, CHUNK), jnp.int32),
        pltpu.VMEM((NCHUNK, CHUNK), jnp.int32),
        pltpu.VMEM((NCHUNK, CHUNK), jnp.int32),
        pltpu.VMEM((NCHUNK, CHUNK), jnp.int32),
        pltpu.SemaphoreType.DMA,
    ],
)
def _sc_dedup(idx_hbm, fw_hbm, tbl, idx_v, pos_v, sidx_v, w_v, sem):
    c = lax.axis_index("c")
    s = lax.axis_index("s")
    base = s * N_TILE
    lane = lax.iota(jnp.int32, L)

    pltpu.sync_copy(idx_hbm.at[pl.ds(s * NCHUNK, NCHUNK)], idx_v)
    for j in range(NCHUNK):
        for k in range(CHUNK // L):
            pos_v[j, pl.ds(k * L, L)] = base + (j * CHUNK + k * L) + lane

    def _scatter_pos(index_ref):
        cps = [pltpu.async_copy(pos_v.at[j], tbl.at[index_ref.at[j]], sem)
               for j in range(NCHUNK)]
        for c_ in cps:
            c_.wait()

    def _gather_w():
        cps = [pltpu.async_copy(tbl.at[idx_v.at[j]], w_v.at[j], sem)
               for j in range(NCHUNK)]
        for c_ in cps:
            c_.wait()

    _scatter_pos(idx_v)
    plsc.subcore_barrier()
    _gather_w()

    for _ in range(ROUNDS):
        for j in range(NCHUNK):
            for k in range(CHUNK // L):
                sl = pl.ds(k * L, L)
                p = pos_v[j, sl]
                w = w_v[j, sl]
                sidx_v[j, sl] = jnp.where(p > w, idx_v[j, sl], DUMMY)
        plsc.subcore_barrier()
        _scatter_pos(sidx_v)
        plsc.subcore_barrier()
        _gather_w()

    @pl.when(c == 0)
    def _():
        pltpu.sync_copy(w_v, fw_hbm.at[pl.ds(s * NCHUNK, NCHUNK)])


def _tc_scatter_body(idx_sm, fw_sm, upd_any, data_any, out_any, sem_arr):
    del data_any

    def _start(i, k):
        pltpu.make_async_copy(
            upd_any.at[pl.ds(fw_sm[i], 1)], out_any.at[pl.ds(idx_sm[i], 1)],
            sem_arr.at[k],
        ).start()

    def _drain(k):
        pltpu.make_async_copy(
            upd_any.at[pl.ds(0, 1)], out_any.at[pl.ds(0, 1)], sem_arr.at[k]
        ).wait()

    for k in range(RING):
        _start(k, k)

    def body(step, carry):
        i0 = RING + step * RING
        for k in range(RING):
            _drain(k)
            _start(i0 + k, k)
        return carry

    lax.fori_loop(0, B // RING - 1, body, 0)
    for k in range(RING):
        _drain(k)


_tc_scatter = pl.pallas_call(
    _tc_scatter_body,
    out_shape=jax.ShapeDtypeStruct((NROWS, D), jnp.float32),
    in_specs=[
        pl.BlockSpec(memory_space=pltpu.SMEM),
        pl.BlockSpec(memory_space=pltpu.SMEM),
        pl.BlockSpec(memory_space=pl.ANY),
        pl.BlockSpec(memory_space=pl.ANY),
    ],
    out_specs=pl.BlockSpec(memory_space=pl.ANY),
    scratch_shapes=[pltpu.SemaphoreType.DMA((RING,))],
    input_output_aliases={3: 0},
)


def kernel(data, indices, updates):
    idx2d = indices.reshape(B).astype(jnp.int32).reshape(B // CHUNK, CHUNK)
    fw2d = _sc_dedup(idx2d)
    idx_flat = idx2d.reshape(B)
    fw_flat = fw2d.reshape(B)
    return _tc_scatter(idx_flat, fw_flat, updates, data)
